# baseline (speedup 1.0000x reference)
import jax
import jax.numpy as jnp
from jax.experimental import pallas as pl
from jax.experimental.pallas import tpu as pltpu

TOKENS = 16384
HIDDEN = 4096
EXPERTS = 64
K = 8

BLOCK_T = 512


def _router_block(x_ref, wt_ref, logits_ref, w_ref, i_ref):
    logits = jax.lax.dot_general(
        x_ref[...].astype(jnp.bfloat16),
        wt_ref[...].astype(jnp.bfloat16),
        dimension_numbers=(((1,), (1,)), ((), ())),
        preferred_element_type=jnp.float32,
    )
    logits_ref[...] = logits

    lt = logits.T
    m = jnp.max(lt, axis=0, keepdims=True)
    e = jnp.exp(lt - m)
    denom = jnp.sum(e, axis=0, keepdims=True)
    p = e / denom

    iota = jax.lax.broadcasted_iota(jnp.int32, lt.shape, 0).astype(jnp.float32)
    cur = p
    vals = []
    idxs = []
    for _ in range(K):
        mx = jnp.max(cur, axis=0, keepdims=True)
        eq = cur == mx
        idxf = jnp.min(jnp.where(eq, iota, float(EXPERTS)), axis=0, keepdims=True)
        vals.append(mx)
        idxs.append(idxf)
        cur = jnp.where(iota == idxf, -1.0, cur)

    wv = jnp.concatenate(vals, axis=0)
    ivf = jnp.concatenate(idxs, axis=0)
    wv = wv / (jnp.sum(wv, axis=0, keepdims=True) + 1e-9)
    w_ref[...] = wv.T
    i_ref[...] = ivf.T.astype(jnp.int32)


@jax.jit
def kernel(x, W):
    wt = W
    grid = (TOKENS // BLOCK_T,)
    out_shapes = (
        jax.ShapeDtypeStruct((TOKENS, EXPERTS), jnp.float32),
        jax.ShapeDtypeStruct((TOKENS, K), jnp.float32),
        jax.ShapeDtypeStruct((TOKENS, K), jnp.int32),
    )
    logits, w, i = pl.pallas_call(
        _router_block,
        grid=grid,
        in_specs=[
            pl.BlockSpec((BLOCK_T, HIDDEN), lambda t: (t, 0)),
            pl.BlockSpec((EXPERTS, HIDDEN), lambda t: (0, 0)),
        ],
        out_specs=(
            pl.BlockSpec((BLOCK_T, EXPERTS), lambda t: (t, 0)),
            pl.BlockSpec((BLOCK_T, K), lambda t: (t, 0)),
            pl.BlockSpec((BLOCK_T, K), lambda t: (t, 0)),
        ),
        out_shape=out_shapes,
        compiler_params=pltpu.CompilerParams(
            dimension_semantics=("parallel",),
        ),
    )(x, wt)
    return (w, i, logits)


# speedup vs baseline: 1.0440x; 1.0440x over previous
import jax
import jax.numpy as jnp
from jax.experimental import pallas as pl
from jax.experimental.pallas import tpu as pltpu

TOKENS = 16384
HIDDEN = 4096
EXPERTS = 64
K = 8

BLOCK_T = 1024


def _router_block(x_ref, wt_ref, logits_ref, w_ref, i_ref):
    logits = jax.lax.dot_general(
        x_ref[...].astype(jnp.bfloat16),
        wt_ref[...].astype(jnp.bfloat16),
        dimension_numbers=(((1,), (1,)), ((), ())),
        preferred_element_type=jnp.float32,
    )
    logits_ref[...] = logits

    lt = logits.T
    m = jnp.max(lt, axis=0, keepdims=True)
    e = jnp.exp(lt - m)
    denom = jnp.sum(e, axis=0, keepdims=True)
    p = e / denom

    iota = jax.lax.broadcasted_iota(jnp.int32, lt.shape, 0).astype(jnp.float32)
    cur = p
    vals = []
    idxs = []
    for _ in range(K):
        mx = jnp.max(cur, axis=0, keepdims=True)
        eq = cur == mx
        idxf = jnp.min(jnp.where(eq, iota, float(EXPERTS)), axis=0, keepdims=True)
        vals.append(mx)
        idxs.append(idxf)
        cur = jnp.where(iota == idxf, -1.0, cur)

    wv = jnp.concatenate(vals, axis=0)
    ivf = jnp.concatenate(idxs, axis=0)
    wv = wv / (jnp.sum(wv, axis=0, keepdims=True) + 1e-9)
    w_ref[...] = wv.T
    i_ref[...] = ivf.T.astype(jnp.int32)


@jax.jit
def kernel(x, W):
    grid = (TOKENS // BLOCK_T,)
    out_shapes = (
        jax.ShapeDtypeStruct((TOKENS, EXPERTS), jnp.float32),
        jax.ShapeDtypeStruct((TOKENS, K), jnp.float32),
        jax.ShapeDtypeStruct((TOKENS, K), jnp.int32),
    )
    logits, w, i = pl.pallas_call(
        _router_block,
        grid=grid,
        in_specs=[
            pl.BlockSpec((BLOCK_T, HIDDEN), lambda t: (t, 0)),
            pl.BlockSpec((EXPERTS, HIDDEN), lambda t: (0, 0)),
        ],
        out_specs=(
            pl.BlockSpec((BLOCK_T, EXPERTS), lambda t: (t, 0)),
            pl.BlockSpec((BLOCK_T, K), lambda t: (t, 0)),
            pl.BlockSpec((BLOCK_T, K), lambda t: (t, 0)),
        ),
        out_shape=out_shapes,
        compiler_params=pltpu.CompilerParams(
            dimension_semantics=("parallel",),
        ),
    )(x, W)
    return (w, i, logits)
